# baseline (speedup 1.0000x reference)
import functools

import jax
import jax.numpy as jnp
from jax import lax
from jax.experimental import pallas as pl
from jax.experimental.pallas import tpu as pltpu
from jax.experimental.pallas import tpu_sc as plsc

_NUM_CLASSES = 100000
_EMBED_DIM = 64
_BATCH = 16384

_NC = 2
_NS = 16
_NW = _NC * _NS
_L = 16
_B_PER_W = _BATCH // _NW


def _center_loss_partials(embeddings, labels, centers):
  mesh = plsc.VectorSubcoreMesh(core_axis_name="c", subcore_axis_name="s")

  @functools.partial(
      pl.kernel,
      mesh=mesh,
      out_type=jax.ShapeDtypeStruct((_NW, _L), jnp.float32),
      compiler_params=pltpu.CompilerParams(use_tc_tiling_on_sc=False),
      scratch_types=[
          pltpu.VMEM((_B_PER_W,), jnp.int32),
          pltpu.VMEM((_B_PER_W, _EMBED_DIM), jnp.float32),
          pltpu.VMEM((_B_PER_W, _EMBED_DIM), jnp.float32),
          pltpu.VMEM((_L,), jnp.float32),
          pltpu.SemaphoreType.DMA,
      ],
  )
  def body(emb_hbm, lab_hbm, cent_hbm, out_hbm, idx_v, cen_v, emb_v, acc_v,
           sem):
    wid = lax.axis_index("s") * _NC + lax.axis_index("c")
    base = wid * _B_PER_W
    pltpu.sync_copy(lab_hbm.at[pl.ds(base, _B_PER_W)], idx_v)
    gather = pltpu.async_copy(cent_hbm.at[idx_v], cen_v, sem)
    pltpu.sync_copy(emb_hbm.at[pl.ds(base, _B_PER_W)], emb_v)
    gather.wait()

    def row_body(i, accs):
      new = []
      for j in range(_EMBED_DIM // _L):
        e = emb_v[i, pl.ds(j * _L, _L)]
        c = cen_v[i, pl.ds(j * _L, _L)]
        d = e - c
        new.append(accs[j] + d * d)
      return tuple(new)

    accs = lax.fori_loop(
        0, _B_PER_W, row_body,
        tuple(jnp.zeros((_L,), jnp.float32) for _ in range(_EMBED_DIM // _L)))
    total = accs[0]
    for j in range(1, _EMBED_DIM // _L):
      total = total + accs[j]
    acc_v[...] = total
    pltpu.sync_copy(acc_v, out_hbm.at[wid])

  return body(embeddings, labels, centers)


def kernel(embeddings, labels, centers):
  partials = _center_loss_partials(embeddings, labels.astype(jnp.int32),
                                   centers)
  return jnp.sum(partials) / _BATCH


# speedup vs baseline: 1.6332x; 1.6332x over previous
import functools

import jax
import jax.numpy as jnp
from jax import lax
from jax.experimental import pallas as pl
from jax.experimental.pallas import tpu as pltpu
from jax.experimental.pallas import tpu_sc as plsc

_NUM_CLASSES = 100000
_EMBED_DIM = 64
_BATCH = 16384

_NC = 2
_NS = 16
_NW = _NC * _NS
_L = 16
_FEATS_PER_W = _EMBED_DIM // _NW
_CHUNK = 4096


def _center_loss_partials(emb_t, labels, cent_t):
  mesh = plsc.VectorSubcoreMesh(core_axis_name="c", subcore_axis_name="s")

  @functools.partial(
      pl.kernel,
      mesh=mesh,
      out_type=jax.ShapeDtypeStruct((_NW, _L), jnp.float32),
      compiler_params=pltpu.CompilerParams(use_tc_tiling_on_sc=True,
                                           needs_layout_passes=False),
      scratch_types=[
          pltpu.VMEM((_NUM_CLASSES,), jnp.float32),
          pltpu.VMEM((_CHUNK,), jnp.int32),
          pltpu.VMEM((_CHUNK,), jnp.float32),
          pltpu.VMEM((_L,), jnp.float32),
          pltpu.SemaphoreType.DMA,
      ],
  )
  def body(emb_hbm, lab_hbm, cent_hbm, out_hbm, crow_v, lab_v, erow_v, acc_v,
           sem):
    wid = lax.axis_index("s") * _NC + lax.axis_index("c")

    acc = jnp.zeros((_L,), jnp.float32)
    for f in range(_FEATS_PER_W):
      c = wid * _FEATS_PER_W + f
      pltpu.sync_copy(cent_hbm.at[c], crow_v)
      for chunk in range(_BATCH // _CHUNK):
        base = chunk * _CHUNK
        pltpu.sync_copy(lab_hbm.at[pl.ds(base, _CHUNK)], lab_v)
        pltpu.sync_copy(emb_hbm.at[c, pl.ds(base, _CHUNK)], erow_v)

        def iter_body(j, a):
          lv = lab_v[pl.ds(j * _L, _L)]
          g = plsc.load_gather(crow_v, [lv])
          e = erow_v[pl.ds(j * _L, _L)]
          d = e - g
          return a + d * d

        acc = lax.fori_loop(0, _CHUNK // _L, iter_body, acc)

    acc_v[...] = acc
    pltpu.sync_copy(acc_v, out_hbm.at[wid])

  return body(emb_t, labels, cent_t)


def kernel(embeddings, labels, centers):
  partials = _center_loss_partials(embeddings.T, labels.astype(jnp.int32),
                                   centers.T)
  return jnp.sum(partials) / _BATCH


# speedup vs baseline: 2.1690x; 1.3280x over previous
import functools

import jax
import jax.numpy as jnp
from jax import lax
from jax.experimental import pallas as pl
from jax.experimental.pallas import tpu as pltpu
from jax.experimental.pallas import tpu_sc as plsc

_NUM_CLASSES = 100000
_EMBED_DIM = 64
_BATCH = 16384

_NC = 2
_NS = 16
_NW = _NC * _NS
_L = 16
_FEATS_PER_W = _EMBED_DIM // _NW
_CHUNK = 4096


def _center_loss_partials(emb_t, labels, cent_t):
  mesh = plsc.VectorSubcoreMesh(core_axis_name="c", subcore_axis_name="s")

  @functools.partial(
      pl.kernel,
      mesh=mesh,
      out_type=jax.ShapeDtypeStruct((_NW, _L), jnp.float32),
      compiler_params=pltpu.CompilerParams(use_tc_tiling_on_sc=True,
                                           needs_layout_passes=False),
      scratch_types=[
          pltpu.VMEM((_NUM_CLASSES,), jnp.float32),
          pltpu.VMEM((2, _CHUNK), jnp.int32),
          pltpu.VMEM((2, _CHUNK), jnp.float32),
          pltpu.VMEM((_L,), jnp.float32),
          pltpu.SemaphoreType.DMA,
          pltpu.SemaphoreType.DMA,
      ],
  )
  def body(emb_hbm, lab_hbm, cent_hbm, out_hbm, crow_v, lab_v, erow_v, acc_v,
           sem_row, sem_chunk):
    wid = lax.axis_index("s") * _NC + lax.axis_index("c")
    n_chunks = _BATCH // _CHUNK
    unroll = 8

    def start_chunk(c, chunk, buf):
      base = chunk * _CHUNK
      lab_cp = pltpu.async_copy(lab_hbm.at[pl.ds(base, _CHUNK)],
                                lab_v.at[buf], sem_chunk)
      erow_cp = pltpu.async_copy(emb_hbm.at[c, pl.ds(base, _CHUNK)],
                                 erow_v.at[buf], sem_chunk)
      return lab_cp, erow_cp

    acc = jnp.zeros((_L,), jnp.float32)
    pending = ()
    for f in range(_FEATS_PER_W):
      c = wid * _FEATS_PER_W + f
      row_cp = pltpu.async_copy(cent_hbm.at[c], crow_v, sem_row)
      if f == 0:
        pending = start_chunk(c, 0, 0)
      row_cp.wait()
      for chunk in range(n_chunks):
        buf = (f * n_chunks + chunk) % 2
        for cp in pending:
          cp.wait()
        if chunk + 1 < n_chunks:
          pending = start_chunk(c, chunk + 1, 1 - buf)
        elif f + 1 < _FEATS_PER_W:
          pending = start_chunk(c + 1, 0, 1 - buf)
        else:
          pending = ()

        def iter_body(j, a, buf=buf):
          base = j * (_L * unroll)
          for u in range(unroll):
            lv = lab_v[buf, pl.ds(base + u * _L, _L)]
            g = plsc.load_gather(crow_v, [lv])
            e = erow_v[buf, pl.ds(base + u * _L, _L)]
            d = e - g
            a = a + d * d
          return a

        acc = lax.fori_loop(0, _CHUNK // (_L * unroll), iter_body, acc)

    acc_v[...] = acc
    pltpu.sync_copy(acc_v, out_hbm.at[wid])

  return body(emb_t, labels, cent_t)


def kernel(embeddings, labels, centers):
  partials = _center_loss_partials(embeddings.T, labels.astype(jnp.int32),
                                   centers.T)
  return jnp.sum(partials) / _BATCH


# speedup vs baseline: 2.1733x; 1.0020x over previous
import functools

import jax
import jax.numpy as jnp
from jax import lax
from jax.experimental import pallas as pl
from jax.experimental.pallas import tpu as pltpu
from jax.experimental.pallas import tpu_sc as plsc

_NUM_CLASSES = 100000
_EMBED_DIM = 64
_BATCH = 16384

_NC = 2
_NS = 16
_NW = _NC * _NS
_L = 16
_FEATS_PER_W = _EMBED_DIM // _NW
_CHUNK = 4096


def _center_loss_partials(emb_t, labels, cent_t):
  mesh = plsc.VectorSubcoreMesh(core_axis_name="c", subcore_axis_name="s")

  @functools.partial(
      pl.kernel,
      mesh=mesh,
      out_type=jax.ShapeDtypeStruct((_NW, _L), jnp.float32),
      compiler_params=pltpu.CompilerParams(use_tc_tiling_on_sc=True,
                                           needs_layout_passes=False),
      scratch_types=[
          pltpu.VMEM((_NUM_CLASSES,), jnp.float32),
          pltpu.VMEM((2, _CHUNK), jnp.int32),
          pltpu.VMEM((2, _CHUNK), jnp.float32),
          pltpu.VMEM((_L,), jnp.float32),
          pltpu.SemaphoreType.DMA,
          pltpu.SemaphoreType.DMA,
      ],
  )
  def body(emb_hbm, lab_hbm, cent_hbm, out_hbm, crow_v, lab_v, erow_v, acc_v,
           sem_row, sem_chunk):
    wid = lax.axis_index("s") * _NC + lax.axis_index("c")
    n_chunks = _BATCH // _CHUNK
    unroll = 8

    def start_chunk(c, chunk, buf):
      base = chunk * _CHUNK
      lab_cp = pltpu.async_copy(lab_hbm.at[pl.ds(base, _CHUNK)],
                                lab_v.at[buf], sem_chunk)
      erow_cp = pltpu.async_copy(emb_hbm.at[c, pl.ds(base, _CHUNK)],
                                 erow_v.at[buf], sem_chunk)
      return lab_cp, erow_cp

    acc = tuple(jnp.zeros((_L,), jnp.float32) for _ in range(4))
    pending = ()
    for f in range(_FEATS_PER_W):
      c = wid * _FEATS_PER_W + f
      row_cp = pltpu.async_copy(cent_hbm.at[c], crow_v, sem_row)
      if f == 0:
        pending = start_chunk(c, 0, 0)
      row_cp.wait()
      for chunk in range(n_chunks):
        buf = (f * n_chunks + chunk) % 2
        for cp in pending:
          cp.wait()
        if chunk + 1 < n_chunks:
          pending = start_chunk(c, chunk + 1, 1 - buf)
        elif f + 1 < _FEATS_PER_W:
          pending = start_chunk(c + 1, 0, 1 - buf)
        else:
          pending = ()

        def iter_body(j, accs, buf=buf):
          base = j * (_L * unroll)
          new = list(accs)
          for u in range(unroll):
            lv = lab_v[buf, pl.ds(base + u * _L, _L)]
            g = plsc.load_gather(crow_v, [lv])
            e = erow_v[buf, pl.ds(base + u * _L, _L)]
            d = e - g
            new[u % 4] = new[u % 4] + d * d
          return tuple(new)

        acc = lax.fori_loop(0, _CHUNK // (_L * unroll), iter_body, acc)

    total = (acc[0] + acc[1]) + (acc[2] + acc[3])
    acc_v[...] = total
    pltpu.sync_copy(acc_v, out_hbm.at[wid])

  return body(emb_t, labels, cent_t)


def kernel(embeddings, labels, centers):
  partials = _center_loss_partials(embeddings.T, labels.astype(jnp.int32),
                                   centers.T)
  return jnp.sum(partials) / _BATCH
